# baseline (speedup 1.0000x reference)
import functools

import jax
import jax.numpy as jnp
from jax import lax
from jax.experimental import pallas as pl
from jax.experimental.pallas import tpu as pltpu
from jax.experimental.pallas import tpu_sc as plsc

_B = 16384
_D = 32
_NC = 2
_NS = 16
_NW = _NC * _NS
_BPW = _B // _NW
_CH = 16
_NCH = _BPW // _CH
_RING = 8
_LANES = 128


def _cf_body(uidx_hbm, iidx_hbm, utab_hbm, itab_hbm, ubias_hbm, ibias_hbm,
             out_hbm, uidx_v, iidx_v, uslab_v, islab_v, p_v, ub_v, ib_v,
             out_v, sems_u, sems_i, sem_bu, sem_bi):
    wid = lax.axis_index("s") * _NC + lax.axis_index("c")
    base = wid * _BPW

    pltpu.sync_copy(uidx_hbm.at[pl.ds(base, _BPW)], uidx_v)
    pltpu.sync_copy(iidx_hbm.at[pl.ds(base, _BPW)], iidx_v)
    cbu = pltpu.async_copy(ubias_hbm.at[uidx_v], ub_v, sem_bu)
    cbi = pltpu.async_copy(ibias_hbm.at[iidx_v], ib_v, sem_bi)
    cbu.wait()
    cbi.wait()

    iota16 = lax.iota(jnp.int32, 16)
    iota16b = iota16 + 16

    def fire(tab_hbm, slab_v, sems, r, slot):
        col = lax.shift_left(lax.shift_right_logical(r, 7), 7)
        col = pl.multiple_of(col, _LANES)
        pltpu.async_copy(tab_hbm.at[:, pl.ds(col, _LANES)],
                         slab_v.at[slot], sems.at[slot])

    def drain(tab_hbm, slab_v, sems, slot):
        pltpu.make_async_copy(tab_hbm.at[:, pl.ds(0, _LANES)],
                              slab_v.at[slot], sems.at[slot]).wait()

    def pick(slab_v, slot, l):
        lv = jnp.broadcast_to(l, (16,))
        lo = plsc.load_gather(slab_v.at[slot], [iota16, lv])
        hi = plsc.load_gather(slab_v.at[slot], [iota16b, lv])
        return lo, hi

    def group(g, _):
        uvec = uidx_v[pl.ds(g * _CH, _CH)]
        ivec = iidx_v[pl.ds(g * _CH, _CH)]
        lanes = []
        for j in range(_CH):
            if j >= _RING:
                slot = j - _RING
                drain(utab_hbm, uslab_v, sems_u, slot)
                drain(itab_hbm, islab_v, sems_i, slot)
                lu, li = lanes[slot]
                u0, u1 = pick(uslab_v, slot, lu)
                i0, i1 = pick(islab_v, slot, li)
                p_v[slot] = u0 * i0 + u1 * i1
            ur = uvec[j]
            ir = ivec[j]
            lanes.append((ur & 127, ir & 127))
            slot = j % _RING
            fire(utab_hbm, uslab_v, sems_u, ur, slot)
            fire(itab_hbm, islab_v, sems_i, ir, slot)
        for j in range(_CH - _RING, _CH):
            slot = j % _RING
            drain(utab_hbm, uslab_v, sems_u, slot)
            drain(itab_hbm, islab_v, sems_i, slot)
            lu, li = lanes[j]
            u0, u1 = pick(uslab_v, slot, lu)
            i0, i1 = pick(islab_v, slot, li)
            p_v[_RING + slot] = u0 * i0 + u1 * i1

        acc = ub_v[pl.ds(g * _CH, _CH)] + ib_v[pl.ds(g * _CH, _CH)]
        for c in range(16):
            cv = jnp.broadcast_to(jnp.int32(c), (16,))
            acc = acc + plsc.load_gather(p_v, [iota16, cv])
        out_v[pl.ds(g * _CH, _CH)] = acc
        return _

    lax.fori_loop(0, _NCH, group, None)
    pltpu.sync_copy(out_v, out_hbm.at[pl.ds(base, _BPW)])


@jax.jit
def _cf_predict(user_indices, item_indices, user_emb_t, item_emb_t,
                user_bias, item_bias):
    mesh = plsc.VectorSubcoreMesh(core_axis_name="c", subcore_axis_name="s")
    f = pl.kernel(
        _cf_body,
        out_type=jax.ShapeDtypeStruct((_B,), jnp.float32),
        mesh=mesh,
        scratch_types=[
            pltpu.VMEM((_BPW,), jnp.int32),
            pltpu.VMEM((_BPW,), jnp.int32),
            pltpu.VMEM((_RING, _D, _LANES), jnp.float32),
            pltpu.VMEM((_RING, _D, _LANES), jnp.float32),
            pltpu.VMEM((_CH, 16), jnp.float32),
            pltpu.VMEM((_BPW,), jnp.float32),
            pltpu.VMEM((_BPW,), jnp.float32),
            pltpu.VMEM((_BPW,), jnp.float32),
            pltpu.SemaphoreType.DMA((_RING,)),
            pltpu.SemaphoreType.DMA((_RING,)),
            pltpu.SemaphoreType.DMA,
            pltpu.SemaphoreType.DMA,
        ],
        compiler_params=pltpu.CompilerParams(
            needs_layout_passes=False, use_tc_tiling_on_sc=False),
    )
    return f(user_indices, item_indices, user_emb_t, item_emb_t,
             user_bias, item_bias)


def kernel(user_indices, item_indices, user_emb_table, item_emb_table,
           user_bias_table, item_bias_table):
    return _cf_predict(user_indices, item_indices, user_emb_table.T,
                       item_emb_table.T, user_bias_table.reshape(-1),
                       item_bias_table.reshape(-1))


# speedup vs baseline: 5.9204x; 5.9204x over previous
import jax
import jax.numpy as jnp
from jax import lax
from jax.experimental import pallas as pl
from jax.experimental.pallas import tpu as pltpu
from jax.experimental.pallas import tpu_sc as plsc

_B = 16384
_D = 32
_NC = 2
_NS = 16
_NW = _NC * _NS
_BPW = _B // _NW
_CH = 16
_NCH = _BPW // _CH


def _cf_body(uidx_hbm, iidx_hbm, utab_hbm, itab_hbm, ubias_hbm, ibias_hbm,
             out_hbm, uidx_v, iidx_v, urows_v, irows_v, ub_v, ib_v, out_v,
             sem_u, sem_i, sem_bu, sem_bi):
    wid = lax.axis_index("s") * _NC + lax.axis_index("c")
    base = wid * _BPW

    pltpu.sync_copy(uidx_hbm.at[pl.ds(base, _BPW)], uidx_v)
    pltpu.sync_copy(iidx_hbm.at[pl.ds(base, _BPW)], iidx_v)

    cu = pltpu.async_copy(utab_hbm.at[uidx_v], urows_v, sem_u)
    ci = pltpu.async_copy(itab_hbm.at[iidx_v], irows_v, sem_i)
    cbu = pltpu.async_copy(ubias_hbm.at[uidx_v], ub_v, sem_bu)
    cbi = pltpu.async_copy(ibias_hbm.at[iidx_v], ib_v, sem_bi)
    cu.wait()
    ci.wait()
    cbu.wait()
    cbi.wait()

    lane = lax.iota(jnp.int32, 16)

    def chunk(c, _):
        rows = lane + c * _CH
        acc = ub_v[pl.ds(c * _CH, _CH)] + ib_v[pl.ds(c * _CH, _CH)]
        for d in range(_D):
            col = jnp.full((16,), d, jnp.int32)
            uc = plsc.load_gather(urows_v, [rows, col])
            ic = plsc.load_gather(irows_v, [rows, col])
            acc = acc + uc * ic
        out_v[pl.ds(c * _CH, _CH)] = acc
        return _

    lax.fori_loop(0, _NCH, chunk, None)
    pltpu.sync_copy(out_v, out_hbm.at[pl.ds(base, _BPW)])


@jax.jit
def _cf_predict(user_indices, item_indices, user_emb_table, item_emb_table,
                user_bias_table, item_bias_table):
    mesh = plsc.VectorSubcoreMesh(core_axis_name="c", subcore_axis_name="s")
    f = pl.kernel(
        _cf_body,
        out_type=jax.ShapeDtypeStruct((_B,), jnp.float32),
        mesh=mesh,
        scratch_types=[
            pltpu.VMEM((_BPW,), jnp.int32),
            pltpu.VMEM((_BPW,), jnp.int32),
            pltpu.VMEM((_BPW, _D), jnp.float32),
            pltpu.VMEM((_BPW, _D), jnp.float32),
            pltpu.VMEM((_BPW,), jnp.float32),
            pltpu.VMEM((_BPW,), jnp.float32),
            pltpu.VMEM((_BPW,), jnp.float32),
            pltpu.SemaphoreType.DMA,
            pltpu.SemaphoreType.DMA,
            pltpu.SemaphoreType.DMA,
            pltpu.SemaphoreType.DMA,
        ],
        compiler_params=pltpu.CompilerParams(
            needs_layout_passes=False, use_tc_tiling_on_sc=False),
    )
    return f(user_indices, item_indices, user_emb_table, item_emb_table,
             user_bias_table, item_bias_table)


def kernel(user_indices, item_indices, user_emb_table, item_emb_table,
           user_bias_table, item_bias_table):
    return _cf_predict(user_indices, item_indices, user_emb_table,
                       item_emb_table, user_bias_table.reshape(-1),
                       item_bias_table.reshape(-1))
